# baseline (speedup 1.0000x reference)
import functools

import jax
import jax.numpy as jnp
from jax import lax
from jax.experimental import pallas as pl
from jax.experimental.pallas import tpu as pltpu
from jax.experimental.pallas import tpu_sc as plsc

N = 10000
D = 128
E = 320000
PROJ = 64

NC = 2
NS = 16
NW = NC * NS
CHUNK = 96
CH_T = 210
HCH = 30
NPART = CH_T // HCH
E_T = CH_T * CHUNK
E_PAD = NS * E_T
N_ACC = 10240
ZROWS = 64
ZREP = N_ACC // (NS * ZROWS)
WB = 632
WB_LAST = N - WB * (NS - 1)

_HIGH = lax.Precision.HIGHEST


def _mm(x, w):
    return jnp.dot(x, w, precision=lax.Precision.DEFAULT, preferred_element_type=jnp.float32)


def _elu(x):
    return jnp.where(x > 0, x, jnp.exp(x) - 1.0)


def _segsum_body(h_hbm, src_hbm, dst_hbm, out_hbm,
                 src_all, dst_all, rows0_v, rows1_v, rows2_v, acc_sh,
                 gsem0, gsem1, gsem2, ssem0, ssem1, ssem2):
    cid = lax.axis_index("c")
    sid = lax.axis_index("s")

    zero16 = jnp.zeros((16,), jnp.float32)

    def _zrow(i, _):
        def _zcol(j, _):
            rows0_v[i, pl.ds(j * 16, 16)] = zero16
            return 0
        return lax.fori_loop(0, D // 16, _zcol, 0)

    lax.fori_loop(0, ZROWS, _zrow, 0)

    for k in range(ZREP):
        pltpu.sync_copy(rows0_v.at[pl.ds(0, ZROWS)],
                        acc_sh.at[pl.ds((sid * ZREP + k) * ZROWS, ZROWS)])
    plsc.subcore_barrier()

    bufs = ((rows0_v, gsem0, ssem0), (rows1_v, gsem1, ssem1),
            (rows2_v, gsem2, ssem2))

    for enc in range(NC):
        @pl.when(cid == enc)
        def _():
            h_enc = h_hbm.at[enc]

            def _gather(j_row, rows, gsem):
                pltpu.async_copy(h_enc.at[src_all.at[j_row]], rows, gsem)

            def _wait_gather(j_row, rows, gsem):
                pltpu.make_async_copy(h_enc.at[src_all.at[j_row]], rows,
                                      gsem).wait()

            def _scatter(j_row, rows, ssem):
                pltpu.async_copy(rows, acc_sh.at[dst_all.at[j_row]], ssem,
                                 add=True)

            def _wait_scatter(rows, ssem):
                pltpu.make_async_copy(rows, acc_sh.at[dst_all.at[0]],
                                      ssem).wait()

            def _part(p, _):
                pltpu.sync_copy(src_hbm.at[enc, sid, p], src_all)
                pltpu.sync_copy(dst_hbm.at[enc, sid, p], dst_all)
                for b in range(2):
                    _gather(b, bufs[b][0], bufs[b][1])

                def _ring(g, _):
                    for b in range(3):
                        j = g * 3 + b
                        rows, gsem, ssem = bufs[b]
                        rows_p, gsem_p, ssem_p = bufs[(b + 2) % 3]
                        _wait_gather(j, rows, gsem)
                        _scatter(j, rows, ssem)

                        @pl.when(j + 2 < HCH)
                        def _():
                            @pl.when(j >= 1)
                            def _():
                                _wait_scatter(rows_p, ssem_p)
                            _gather(j + 2, rows_p, gsem_p)
                    return 0

                lax.fori_loop(0, HCH // 3, _ring, 0)
                for b in range(3):
                    _wait_scatter(bufs[b][0], bufs[b][2])
                return 0

            lax.fori_loop(0, NPART, _part, 0)
    plsc.subcore_barrier()

    @pl.when(sid < NS - 1)
    def _():
        pltpu.sync_copy(acc_sh.at[pl.ds(sid * WB, WB)],
                        out_hbm.at[cid, pl.ds(sid * WB, WB)])

    @pl.when(sid == NS - 1)
    def _():
        pltpu.sync_copy(acc_sh.at[pl.ds((NS - 1) * WB, WB_LAST)],
                        out_hbm.at[cid, pl.ds((NS - 1) * WB, WB_LAST)])


_segsum = pl.kernel(
    _segsum_body,
    out_type=jax.ShapeDtypeStruct((NC, N, D), jnp.float32),
    mesh=plsc.VectorSubcoreMesh(core_axis_name="c", subcore_axis_name="s"),
    scratch_types=[
        pltpu.VMEM((HCH, CHUNK), jnp.int32),
        pltpu.VMEM((HCH, CHUNK), jnp.int32),
        pltpu.VMEM((CHUNK, D), jnp.float32),
        pltpu.VMEM((CHUNK, D), jnp.float32),
        pltpu.VMEM((CHUNK, D), jnp.float32),
        pltpu.VMEM_SHARED((N_ACC, D), jnp.float32),
        pltpu.SemaphoreType.DMA,
        pltpu.SemaphoreType.DMA,
        pltpu.SemaphoreType.DMA,
        pltpu.SemaphoreType.DMA,
        pltpu.SemaphoreType.DMA,
        pltpu.SemaphoreType.DMA,
    ],
)


def _pad_edges(ei):
    pad = E_PAD - E
    r = jnp.arange(pad, dtype=jnp.int32)
    pad_src = (r * 97) % N
    pad_dst = N + r % (N_ACC - N)
    src = jnp.concatenate([ei[0], pad_src]).reshape(NS, NPART, HCH, CHUNK)
    dst = jnp.concatenate([ei[1], pad_dst]).reshape(NS, NPART, HCH, CHUNK)
    return src, dst


NB = 10
BLK = N // NB

_row_spec = pl.BlockSpec((BLK, D), lambda i: (i, 0))
_w_spec = pl.BlockSpec((D, D), lambda i: (0, 0))
_b_spec = pl.BlockSpec((1, D), lambda i: (0, 0))


def _linear_body(x_ref, w_ref, b_ref, o_ref):
    o_ref[...] = _mm(x_ref[...], w_ref[...]) + b_ref[...]


_linear_call = pl.pallas_call(
    _linear_body,
    grid=(NB,),
    in_specs=[_row_spec, _w_spec, _b_spec],
    out_specs=_row_spec,
    out_shape=jax.ShapeDtypeStruct((N, D), jnp.float32),
)


def _linear(x, p):
    return _linear_call(x, p["W"], p["b"].reshape(1, D))


def _gin_body(scal_ref, h_ref, a_ref, w1_ref, b1_ref, w2_ref, b2_ref,
              g_ref, be_ref, o_ref):
    h = h_ref[...]
    z = scal_ref[0] * h + a_ref[0]
    t = _elu(_mm(z, w1_ref[...]) + b1_ref[...])
    t = _elu(_mm(t, w2_ref[...]) + b2_ref[...])
    y = h + t
    mu = jnp.mean(y, axis=-1, keepdims=True)
    yc = y - mu
    var = jnp.mean(yc * yc, axis=-1, keepdims=True)
    o_ref[...] = yc * lax.rsqrt(var + 1e-5) * g_ref[...] + be_ref[...]


def _make_gin_call(enc):
    return pl.pallas_call(
        _gin_body,
        grid=(NB,),
        in_specs=[
            pl.BlockSpec(memory_space=pltpu.SMEM),
            _row_spec,
            pl.BlockSpec((1, BLK, D), lambda i, e=enc: (e, i, 0)),
            _w_spec, _b_spec, _w_spec, _b_spec, _b_spec, _b_spec,
        ],
        out_specs=_row_spec,
        out_shape=jax.ShapeDtypeStruct((N, D), jnp.float32),
    )


_gin_calls = (_make_gin_call(0), _make_gin_call(1))


def _gin(enc, h, agg2, lp):
    scal = (1.0 + lp["eps"]).reshape(1)
    return _gin_calls[enc](scal, h, agg2,
                     lp["mlp1"]["W"], lp["mlp1"]["b"].reshape(1, D),
                     lp["mlp2"]["W"], lp["mlp2"]["b"].reshape(1, D),
                     lp["ln_g"].reshape(1, D), lp["ln_b"].reshape(1, D))


def _node_body(h_ref, wn_ref, bn_ref, w1_ref, b1_ref, w2_ref, b2_ref,
               o_ref, g_ref):
    i = pl.program_id(0)
    h = h_ref[...]
    ne = _mm(h, wn_ref[...]) + bn_ref[...]
    t = _elu(_mm(ne, w1_ref[...]) + b1_ref[...])
    y = _mm(t, w2_ref[...]) + b2_ref[...]
    nrm = jnp.sqrt(jnp.sum(y * y, axis=-1, keepdims=True))
    o_ref[...] = y / jnp.maximum(nrm, 1e-12)

    s = jnp.sum(h, axis=0, keepdims=True)

    @pl.when(i == 0)
    def _():
        g_ref[...] = s

    @pl.when(i > 0)
    def _():
        g_ref[...] += s


_node_call = pl.pallas_call(
    _node_body,
    grid=(NB,),
    in_specs=[
        _row_spec, _w_spec, _b_spec, _w_spec, _b_spec,
        pl.BlockSpec((D, PROJ), lambda i: (0, 0)),
        pl.BlockSpec((1, PROJ), lambda i: (0, 0)),
    ],
    out_specs=[
        pl.BlockSpec((BLK, PROJ), lambda i: (i, 0)),
        pl.BlockSpec((1, D), lambda i: (0, 0)),
    ],
    out_shape=[
        jax.ShapeDtypeStruct((N, PROJ), jnp.float32),
        jax.ShapeDtypeStruct((1, D), jnp.float32),
    ],
)


def _node_head(h, pn, ph):
    return _node_call(h, pn["W"], pn["b"].reshape(1, D),
                      ph["l1"]["W"], ph["l1"]["b"].reshape(1, D),
                      ph["l2"]["W"], ph["l2"]["b"].reshape(1, PROJ))


def _graph_body(gs_ref, g1w_ref, g1b_ref, g2w_ref, g2b_ref,
                l1w_ref, l1b_ref, l2w_ref, l2b_ref, o_ref):
    for e in range(2):
        g = gs_ref[e:e + 1, :] * (1.0 / N)
        t = _elu(_mm(g, g1w_ref[e]) + g1b_ref[e:e + 1, :])
        ge = _mm(t, g2w_ref[e]) + g2b_ref[e:e + 1, :]
        t = _elu(_mm(ge, l1w_ref[e]) + l1b_ref[e:e + 1, :])
        y = _mm(t, l2w_ref[e]) + l2b_ref[e:e + 1, :]
        nrm = jnp.sqrt(jnp.sum(y * y, axis=-1, keepdims=True))
        o_ref[e:e + 1, :] = y / jnp.maximum(nrm, 1e-12)


_graph_call = pl.pallas_call(
    _graph_body,
    out_shape=jax.ShapeDtypeStruct((2, PROJ), jnp.float32),
)


def kernel(sc_x, fc_x, params, sc_edge_index, fc_edge_index):
    z_node = {}
    gsum = {}
    names = ("sc", "fc")
    src_p, dst_p = {}, {}
    h = {}
    for name, x, ei in ((names[0], sc_x, sc_edge_index),
                        (names[1], fc_x, fc_edge_index)):
        src_p[name], dst_p[name] = _pad_edges(ei)
        h[name] = _linear(x, params[name + "_enc"]["input_proj"])
    src = jnp.stack([src_p["sc"], src_p["fc"]])
    dst = jnp.stack([dst_p["sc"], dst_p["fc"]])

    for li in range(3):
        hs = jnp.stack([h["sc"], h["fc"]])
        agg2 = _segsum(hs, src, dst)
        for enc, name in enumerate(names):
            lp = params[name + "_enc"]["layers"][li]
            h[name] = _gin(enc, h[name], agg2, lp)

    for name in names:
        enc_p = params[name + "_enc"]
        z_node[name], gsum[name] = _node_head(
            h[name], enc_p["node_proj"], params[name + "_node_proj"])

    def stk(fn):
        return jnp.stack([fn("sc"), fn("fc")])

    gs = jnp.concatenate([gsum["sc"], gsum["fc"]], axis=0)
    zg = _graph_call(
        gs,
        stk(lambda n: params[n + "_enc"]["graph_proj1"]["W"]),
        stk(lambda n: params[n + "_enc"]["graph_proj1"]["b"].reshape(1, D)[0]),
        stk(lambda n: params[n + "_enc"]["graph_proj2"]["W"]),
        stk(lambda n: params[n + "_enc"]["graph_proj2"]["b"].reshape(1, D)[0]),
        stk(lambda n: params[n + "_proj"]["l1"]["W"]),
        stk(lambda n: params[n + "_proj"]["l1"]["b"]),
        stk(lambda n: params[n + "_proj"]["l2"]["W"]),
        stk(lambda n: params[n + "_proj"]["l2"]["b"]),
    )
    return (zg[0:1], zg[1:2], z_node["sc"], z_node["fc"])


# speedup vs baseline: 1.0655x; 1.0655x over previous
import functools

import jax
import jax.numpy as jnp
from jax import lax
from jax.experimental import pallas as pl
from jax.experimental.pallas import tpu as pltpu
from jax.experimental.pallas import tpu_sc as plsc

N = 10000
D = 128
E = 320000
PROJ = 64

NC = 2
NS = 16
NW = NC * NS
CHUNK = 96
CH_PER_W = 108
HCH = 36
NPART = CH_PER_W // HCH
E_W = CH_PER_W * CHUNK
E_PAD = NW * E_W
N_ACC = 10240
ZROWS = 64
ZREP = N_ACC // (NS * ZROWS)
WB = 632
WB_LAST = N - WB * (NS - 1)

_HIGH = lax.Precision.HIGHEST


def _mm(x, w):
    return jnp.dot(x, w, precision=lax.Precision.DEFAULT, preferred_element_type=jnp.float32)


def _elu(x):
    return jnp.where(x > 0, x, jnp.exp(x) - 1.0)


def _segsum_body(h_hbm, src_hbm, dst_hbm, out_hbm,
                 src_all, dst_all, rows0_v, rows1_v, rows2_v, acc_sh,
                 gsem0, gsem1, gsem2, ssem0, ssem1, ssem2):
    cid = lax.axis_index("c")
    sid = lax.axis_index("s")
    wid = sid * NC + cid

    zero16 = jnp.zeros((16,), jnp.float32)

    def _zrow(i, _):
        def _zcol(j, _):
            rows0_v[i, pl.ds(j * 16, 16)] = zero16
            return 0
        return lax.fori_loop(0, D // 16, _zcol, 0)

    lax.fori_loop(0, ZROWS, _zrow, 0)

    for k in range(ZREP):
        pltpu.sync_copy(rows0_v.at[pl.ds(0, ZROWS)],
                        acc_sh.at[pl.ds((sid * ZREP + k) * ZROWS, ZROWS)])
    plsc.subcore_barrier()

    bufs = ((rows0_v, gsem0, ssem0), (rows1_v, gsem1, ssem1),
            (rows2_v, gsem2, ssem2))

    def _gather(j_row, rows, gsem):
        pltpu.async_copy(h_hbm.at[src_all.at[j_row]], rows, gsem)

    def _wait_gather(j_row, rows, gsem):
        pltpu.make_async_copy(h_hbm.at[src_all.at[j_row]], rows, gsem).wait()

    def _scatter(j_row, rows, ssem):
        pltpu.async_copy(rows, acc_sh.at[dst_all.at[j_row]], ssem, add=True)

    def _wait_scatter(rows, ssem):
        pltpu.make_async_copy(rows, acc_sh.at[dst_all.at[0]], ssem).wait()

    for part in range(NPART):
        pltpu.sync_copy(src_hbm.at[wid, part], src_all)
        pltpu.sync_copy(dst_hbm.at[wid, part], dst_all)
        for b in range(2):
            _gather(b, bufs[b][0], bufs[b][1])

        def _ring(g, _):
            for b in range(3):
                j = g * 3 + b
                rows, gsem, ssem = bufs[b]
                rows_p, gsem_p, ssem_p = bufs[(b + 2) % 3]
                _wait_gather(j, rows, gsem)
                _scatter(j, rows, ssem)

                @pl.when(j + 2 < HCH)
                def _():
                    @pl.when(j >= 1)
                    def _():
                        _wait_scatter(rows_p, ssem_p)
                    _gather(j + 2, rows_p, gsem_p)
            return 0

        lax.fori_loop(0, HCH // 3, _ring, 0)
        for b in range(3):
            _wait_scatter(bufs[b][0], bufs[b][2])
    plsc.subcore_barrier()

    @pl.when(sid < NS - 1)
    def _():
        pltpu.sync_copy(acc_sh.at[pl.ds(sid * WB, WB)],
                        out_hbm.at[cid, pl.ds(sid * WB, WB)])

    @pl.when(sid == NS - 1)
    def _():
        pltpu.sync_copy(acc_sh.at[pl.ds((NS - 1) * WB, WB_LAST)],
                        out_hbm.at[cid, pl.ds((NS - 1) * WB, WB_LAST)])


_segsum = pl.kernel(
    _segsum_body,
    out_type=jax.ShapeDtypeStruct((NC, N, D), jnp.float32),
    mesh=plsc.VectorSubcoreMesh(core_axis_name="c", subcore_axis_name="s"),
    scratch_types=[
        pltpu.VMEM((HCH, CHUNK), jnp.int32),
        pltpu.VMEM((HCH, CHUNK), jnp.int32),
        pltpu.VMEM((CHUNK, D), jnp.float32),
        pltpu.VMEM((CHUNK, D), jnp.float32),
        pltpu.VMEM((CHUNK, D), jnp.float32),
        pltpu.VMEM_SHARED((N_ACC, D), jnp.float32),
        pltpu.SemaphoreType.DMA,
        pltpu.SemaphoreType.DMA,
        pltpu.SemaphoreType.DMA,
        pltpu.SemaphoreType.DMA,
        pltpu.SemaphoreType.DMA,
        pltpu.SemaphoreType.DMA,
    ],
)


def _pad_edges(ei):
    pad = E_PAD - E
    r = jnp.arange(pad, dtype=jnp.int32)
    pad_src = (r * 97) % N
    pad_dst = N + r % (N_ACC - N)
    src = jnp.concatenate([ei[0], pad_src]).reshape(NW, NPART, HCH, CHUNK)
    dst = jnp.concatenate([ei[1], pad_dst]).reshape(NW, NPART, HCH, CHUNK)
    return src, dst


NB = 10
BLK = N // NB

_row_spec = pl.BlockSpec((BLK, D), lambda i: (i, 0))
_w_spec = pl.BlockSpec((D, D), lambda i: (0, 0))
_b_spec = pl.BlockSpec((1, D), lambda i: (0, 0))


def _linear_body(x_ref, w_ref, b_ref, o_ref):
    o_ref[...] = _mm(x_ref[...], w_ref[...]) + b_ref[...]


_linear_call = pl.pallas_call(
    _linear_body,
    grid=(NB,),
    in_specs=[_row_spec, _w_spec, _b_spec],
    out_specs=_row_spec,
    out_shape=jax.ShapeDtypeStruct((N, D), jnp.float32),
)


def _linear(x, p):
    return _linear_call(x, p["W"], p["b"].reshape(1, D))


def _gin_body(scal_ref, h_ref, a_ref, w1_ref, b1_ref, w2_ref, b2_ref,
              g_ref, be_ref, o_ref):
    h = h_ref[...]
    z = scal_ref[0] * h + a_ref[0] + a_ref[1]
    t = _elu(_mm(z, w1_ref[...]) + b1_ref[...])
    t = _elu(_mm(t, w2_ref[...]) + b2_ref[...])
    y = h + t
    mu = jnp.mean(y, axis=-1, keepdims=True)
    yc = y - mu
    var = jnp.mean(yc * yc, axis=-1, keepdims=True)
    o_ref[...] = yc * lax.rsqrt(var + 1e-5) * g_ref[...] + be_ref[...]


_gin_call = pl.pallas_call(
    _gin_body,
    grid=(NB,),
    in_specs=[
        pl.BlockSpec(memory_space=pltpu.SMEM),
        _row_spec,
        pl.BlockSpec((NC, BLK, D), lambda i: (0, i, 0)),
        _w_spec, _b_spec, _w_spec, _b_spec, _b_spec, _b_spec,
    ],
    out_specs=_row_spec,
    out_shape=jax.ShapeDtypeStruct((N, D), jnp.float32),
)


def _gin(h, agg2, lp):
    scal = (1.0 + lp["eps"]).reshape(1)
    return _gin_call(scal, h, agg2,
                     lp["mlp1"]["W"], lp["mlp1"]["b"].reshape(1, D),
                     lp["mlp2"]["W"], lp["mlp2"]["b"].reshape(1, D),
                     lp["ln_g"].reshape(1, D), lp["ln_b"].reshape(1, D))


def _node_body(h_ref, wn_ref, bn_ref, w1_ref, b1_ref, w2_ref, b2_ref,
               o_ref, g_ref):
    i = pl.program_id(0)
    h = h_ref[...]
    ne = _mm(h, wn_ref[...]) + bn_ref[...]
    t = _elu(_mm(ne, w1_ref[...]) + b1_ref[...])
    y = _mm(t, w2_ref[...]) + b2_ref[...]
    nrm = jnp.sqrt(jnp.sum(y * y, axis=-1, keepdims=True))
    o_ref[...] = y / jnp.maximum(nrm, 1e-12)

    s = jnp.sum(h, axis=0, keepdims=True)

    @pl.when(i == 0)
    def _():
        g_ref[...] = s

    @pl.when(i > 0)
    def _():
        g_ref[...] += s


_node_call = pl.pallas_call(
    _node_body,
    grid=(NB,),
    in_specs=[
        _row_spec, _w_spec, _b_spec, _w_spec, _b_spec,
        pl.BlockSpec((D, PROJ), lambda i: (0, 0)),
        pl.BlockSpec((1, PROJ), lambda i: (0, 0)),
    ],
    out_specs=[
        pl.BlockSpec((BLK, PROJ), lambda i: (i, 0)),
        pl.BlockSpec((1, D), lambda i: (0, 0)),
    ],
    out_shape=[
        jax.ShapeDtypeStruct((N, PROJ), jnp.float32),
        jax.ShapeDtypeStruct((1, D), jnp.float32),
    ],
)


def _node_head(h, pn, ph):
    return _node_call(h, pn["W"], pn["b"].reshape(1, D),
                      ph["l1"]["W"], ph["l1"]["b"].reshape(1, D),
                      ph["l2"]["W"], ph["l2"]["b"].reshape(1, PROJ))


def _graph_body(gs_ref, g1w_ref, g1b_ref, g2w_ref, g2b_ref,
                l1w_ref, l1b_ref, l2w_ref, l2b_ref, o_ref):
    for e in range(2):
        g = gs_ref[e:e + 1, :] * (1.0 / N)
        t = _elu(_mm(g, g1w_ref[e]) + g1b_ref[e:e + 1, :])
        ge = _mm(t, g2w_ref[e]) + g2b_ref[e:e + 1, :]
        t = _elu(_mm(ge, l1w_ref[e]) + l1b_ref[e:e + 1, :])
        y = _mm(t, l2w_ref[e]) + l2b_ref[e:e + 1, :]
        nrm = jnp.sqrt(jnp.sum(y * y, axis=-1, keepdims=True))
        o_ref[e:e + 1, :] = y / jnp.maximum(nrm, 1e-12)


_graph_call = pl.pallas_call(
    _graph_body,
    out_shape=jax.ShapeDtypeStruct((2, PROJ), jnp.float32),
)


def kernel(sc_x, fc_x, params, sc_edge_index, fc_edge_index):
    z_node = {}
    gsum = {}
    names = ("sc", "fc")
    edges = {"sc": _pad_edges(sc_edge_index), "fc": _pad_edges(fc_edge_index)}
    h = {"sc": _linear(sc_x, params["sc_enc"]["input_proj"]),
         "fc": _linear(fc_x, params["fc_enc"]["input_proj"])}
    for li in range(3):
        agg = {}
        for name in names:
            agg[name] = _segsum(h[name], *edges[name])
        for name in names:
            h[name] = _gin(h[name], agg[name],
                           params[name + "_enc"]["layers"][li])
    for name in names:
        z_node[name], gsum[name] = _node_head(
            h[name], params[name + "_enc"]["node_proj"],
            params[name + "_node_proj"])

    def stk(fn):
        return jnp.stack([fn("sc"), fn("fc")])

    gs = jnp.concatenate([gsum["sc"], gsum["fc"]], axis=0)
    zg = _graph_call(
        gs,
        stk(lambda n: params[n + "_enc"]["graph_proj1"]["W"]),
        stk(lambda n: params[n + "_enc"]["graph_proj1"]["b"].reshape(1, D)[0]),
        stk(lambda n: params[n + "_enc"]["graph_proj2"]["W"]),
        stk(lambda n: params[n + "_enc"]["graph_proj2"]["b"].reshape(1, D)[0]),
        stk(lambda n: params[n + "_proj"]["l1"]["W"]),
        stk(lambda n: params[n + "_proj"]["l1"]["b"]),
        stk(lambda n: params[n + "_proj"]["l2"]["W"]),
        stk(lambda n: params[n + "_proj"]["l2"]["b"]),
    )
    return (zg[0:1], zg[1:2], z_node["sc"], z_node["fc"])
